# baseline (speedup 1.0000x reference)
import jax
import jax.numpy as jnp
import numpy as np
from jax import lax
from jax.experimental import pallas as pl
from jax.experimental.pallas import tpu as pltpu
from jax.experimental.pallas import tpu_sc as plsc

D_MODEL = 64
DPAD = 128
SEQ_LEN = 200
BATCH = 1024
XPAD = 256

NUM_CORES = 2
NUM_SUBCORES = 16
NUM_WORKERS = NUM_CORES * NUM_SUBCORES

SEQ_PER_W = BATCH // NUM_WORKERS
G0 = 128
G1 = SEQ_LEN - G0

NBUF = 4
LOOK = 2


def _positional_encoding() -> np.ndarray:
    pe = np.zeros((SEQ_LEN, D_MODEL), dtype=np.float32)
    pos = np.arange(0, SEQ_LEN, dtype=np.float32)[:, None]
    _2i = np.arange(0, D_MODEL, 2, dtype=np.float32)
    pe[:, 0::2] = np.sin(pos / (10000.0 ** (_2i / D_MODEL)))
    pe[:, 1::2] = np.cos(pos / (10000.0 ** (_2i / D_MODEL)))
    return pe


_PE_CONST = _positional_encoding()


def _body(x_hbm, pe_hbm, table_hbm, out_hbm, idx_v, pe_v, rows_v, gsem, ssem,
          isem):
    wid = lax.axis_index("s") * NUM_CORES + lax.axis_index("c")
    seq0 = wid * SEQ_PER_W

    for u in range(SEQ_PER_W):
        pltpu.async_copy(x_hbm.at[seq0 + u],
                         idx_v.at[pl.ds(u * XPAD, XPAD)], isem)
    pltpu.sync_copy(pe_hbm, pe_v)
    for u in range(SEQ_PER_W):
        pltpu.make_async_copy(x_hbm.at[seq0],
                              idx_v.at[pl.ds(0, XPAD)], isem).wait()

    def start_gather(u, b):
        off = u * XPAD
        pltpu.async_copy(
            table_hbm.at[idx_v.at[pl.ds(off, G0)]],
            rows_v.at[b, pl.ds(0, G0)], gsem.at[b])
        pltpu.async_copy(
            table_hbm.at[idx_v.at[pl.ds(off + G0, G1)]],
            rows_v.at[b, pl.ds(G0, G1)], gsem.at[b])

    def wait_gather(b):
        pltpu.make_async_copy(
            table_hbm.at[idx_v.at[pl.ds(0, G0)]],
            rows_v.at[b], gsem.at[b]).wait()

    def start_store(u, b):
        pltpu.async_copy(rows_v.at[b, :, pl.ds(0, D_MODEL)],
                         out_hbm.at[seq0 + u], ssem.at[b])

    def wait_store(b):
        pltpu.make_async_copy(rows_v.at[b, :, pl.ds(0, D_MODEL)],
                              out_hbm.at[seq0], ssem.at[b]).wait()

    for u in range(LOOK):
        start_gather(u, u)

    @pl.loop(0, SEQ_PER_W // NBUF)
    def _group(g):
        for b in range(NBUF):
            u = g * NBUF + b
            b2 = (b + LOOK) % NBUF

            @pl.when(u + LOOK < SEQ_PER_W)
            def _():
                @pl.when(u + LOOK >= NBUF)
                def _():
                    wait_store(b2)
                start_gather(u + LOOK, b2)

            wait_gather(b)

            @pl.loop(0, SEQ_LEN, unroll=4)
            def _row(i):
                for j in range(D_MODEL // 16):
                    sl = pl.ds(j * 16, 16)
                    rows_v[b, i, sl] += pe_v[i, sl]

            start_store(u, b)

    for b in range(NBUF):
        wait_store(b)


@jax.jit
def _embed(x_p, pe, table_p):
    kfn = pl.kernel(
        _body,
        name="embed_gather",
        out_type=jax.ShapeDtypeStruct((BATCH, SEQ_LEN, D_MODEL), jnp.float32),
        mesh=plsc.VectorSubcoreMesh(core_axis_name="c", subcore_axis_name="s"),
        scratch_types=[
            pltpu.VMEM((SEQ_PER_W * XPAD,), jnp.int32),
            pltpu.VMEM((SEQ_LEN, D_MODEL), jnp.float32),
            pltpu.VMEM((NBUF, SEQ_LEN, DPAD), jnp.float32),
            pltpu.SemaphoreType.DMA((NBUF,)),
            pltpu.SemaphoreType.DMA((NBUF,)),
            pltpu.SemaphoreType.DMA,
        ],
        compiler_params=pltpu.CompilerParams(use_tc_tiling_on_sc=False),
    )
    return kfn(x_p, pe, table_p)


def kernel(x, table):
    pe = jnp.asarray(_PE_CONST)
    x_p = jnp.pad(x.astype(jnp.int32), ((0, 0), (0, XPAD - SEQ_LEN)))
    table_p = jnp.pad(table, ((0, 0), (0, DPAD - D_MODEL)))
    return _embed(x_p, pe, table_p)


# speedup vs baseline: 1.1466x; 1.1466x over previous
import jax
import jax.numpy as jnp
import numpy as np
from jax import lax
from jax.experimental import pallas as pl
from jax.experimental.pallas import tpu as pltpu
from jax.experimental.pallas import tpu_sc as plsc

D_MODEL = 64
DPAD = 128
SEQ_LEN = 200
BATCH = 1024
XPAD = 256

NUM_CORES = 2
NUM_SUBCORES = 16
NUM_WORKERS = NUM_CORES * NUM_SUBCORES

SEQ_PER_W = BATCH // NUM_WORKERS
LEN_A = 96
LEN_B = SEQ_LEN - LEN_A
NCHUNK = 2 * SEQ_PER_W

NROWB = 4
NDENB = 2
LOOK = 2


def _positional_encoding_padded() -> np.ndarray:
    pe = np.zeros((SEQ_LEN, DPAD), dtype=np.float32)
    pos = np.arange(0, SEQ_LEN, dtype=np.float32)[:, None]
    _2i = np.arange(0, D_MODEL, 2, dtype=np.float32)
    pe[:, 0:D_MODEL:2] = np.sin(pos / (10000.0 ** (_2i / D_MODEL)))
    pe[:, 1:D_MODEL:2] = np.cos(pos / (10000.0 ** (_2i / D_MODEL)))
    return pe


_PE_CONST = _positional_encoding_padded()


def _chunk_params(c):
    return c // 2, LEN_A * (c % 2), LEN_A if c % 2 == 0 else LEN_B


def _body(x_hbm, pe_hbm, table_hbm, out_hbm, idx_v, pe_v, rows_v, den_v,
          gsem, ssem, isem):
    wid = lax.axis_index("s") * NUM_CORES + lax.axis_index("c")
    seq0 = wid * SEQ_PER_W

    for u in range(SEQ_PER_W):
        pltpu.async_copy(x_hbm.at[seq0 + u, pl.ds(0, 128)],
                         idx_v.at[pl.ds(u * XPAD, 128)], isem)
        pltpu.async_copy(x_hbm.at[seq0 + u, pl.ds(128, 128)],
                         idx_v.at[pl.ds(u * XPAD + 128, 128)], isem)
    pltpu.sync_copy(pe_hbm, pe_v)
    for _ in range(2 * SEQ_PER_W):
        pltpu.make_async_copy(x_hbm.at[seq0, pl.ds(0, 128)],
                              idx_v.at[pl.ds(0, 128)], isem).wait()

    SEQ_PER_GROUP = NROWB // 2

    def start_gather(c, g, b):
        u, s0, ln = _chunk_params(c)
        pltpu.async_copy(
            table_hbm.at[idx_v.at[pl.ds((g * SEQ_PER_GROUP + u) * XPAD + s0,
                                        ln)]],
            rows_v.at[b, pl.ds(0, ln)], gsem.at[b])

    def wait_gather(c, b):
        _, _, ln = _chunk_params(c)
        pltpu.make_async_copy(
            table_hbm.at[idx_v.at[pl.ds(0, ln)]],
            rows_v.at[b, pl.ds(0, ln)], gsem.at[b]).wait()

    def start_store(c, g, d):
        u, s0, ln = _chunk_params(c)
        pltpu.async_copy(den_v.at[d, pl.ds(0, ln)],
                         out_hbm.at[seq0 + g * SEQ_PER_GROUP + u,
                                    pl.ds(s0, ln)], ssem.at[d])

    def wait_store(c, d):
        _, s0, ln = _chunk_params(c)
        pltpu.make_async_copy(den_v.at[d, pl.ds(0, ln)],
                              out_hbm.at[seq0, pl.ds(s0, ln)],
                              ssem.at[d]).wait()

    for c in range(LOOK):
        start_gather(c, 0, c)

    @pl.loop(0, NCHUNK // NROWB)
    def _group(g):
        for b in range(NROWB):
            c = b
            b2 = (b + LOOK) % NROWB
            d = b % NDENB

            @pl.when(g * NROWB + b + LOOK < NCHUNK)
            def _():
                start_gather(b + LOOK, g, b2)

            wait_gather(c, b)

            @pl.when(g * NROWB + b >= NDENB)
            def _():
                wait_store(c, d)

            u, s0, ln = _chunk_params(c)

            @pl.loop(0, ln, unroll=4)
            def _row(i):
                for j in range(D_MODEL // 16):
                    sl = pl.ds(j * 16, 16)
                    den_v[d, i, sl] = rows_v[b, i, sl] + pe_v[s0 + i, sl]

            start_store(c, g, d)

    for d in range(NDENB):
        wait_store(d, d)


@jax.jit
def _embed(x_p, pe, table_p):
    kfn = pl.kernel(
        _body,
        name="embed_gather",
        out_type=jax.ShapeDtypeStruct((BATCH, SEQ_LEN, D_MODEL), jnp.float32),
        mesh=plsc.VectorSubcoreMesh(core_axis_name="c", subcore_axis_name="s"),
        scratch_types=[
            pltpu.VMEM((SEQ_PER_W * XPAD,), jnp.int32),
            pltpu.VMEM((SEQ_LEN, DPAD), jnp.float32),
            pltpu.VMEM((NROWB, LEN_B, DPAD), jnp.float32),
            pltpu.VMEM((NDENB, LEN_B, D_MODEL), jnp.float32),
            pltpu.SemaphoreType.DMA((NROWB,)),
            pltpu.SemaphoreType.DMA((NDENB,)),
            pltpu.SemaphoreType.DMA,
        ],
    )
    return kfn(x_p, pe, table_p)


def kernel(x, table):
    pe = jnp.asarray(_PE_CONST)
    x_p = jnp.pad(x.astype(jnp.int32), ((0, 0), (0, XPAD - SEQ_LEN)))
    table_p = jnp.pad(table, ((0, 0), (0, DPAD - D_MODEL)))
    return _embed(x_p, pe, table_p)


# speedup vs baseline: 1.1662x; 1.0171x over previous
import jax
import jax.numpy as jnp
import numpy as np
from jax import lax
from jax.experimental import pallas as pl
from jax.experimental.pallas import tpu as pltpu
from jax.experimental.pallas import tpu_sc as plsc

D_MODEL = 64
DPAD = 128
SEQ_LEN = 200
BATCH = 1024
XPAD = 256

NUM_CORES = 2
NUM_SUBCORES = 16
NUM_WORKERS = NUM_CORES * NUM_SUBCORES

SEQ_PER_W = BATCH // NUM_WORKERS
LEN_A = 96
LEN_B = SEQ_LEN - LEN_A
NCHUNK = 2 * SEQ_PER_W

NROWB = 4
NDENB = 2
LOOK = 3


def _positional_encoding_padded() -> np.ndarray:
    pe = np.zeros((SEQ_LEN, DPAD), dtype=np.float32)
    pos = np.arange(0, SEQ_LEN, dtype=np.float32)[:, None]
    _2i = np.arange(0, D_MODEL, 2, dtype=np.float32)
    pe[:, 0:D_MODEL:2] = np.sin(pos / (10000.0 ** (_2i / D_MODEL)))
    pe[:, 1:D_MODEL:2] = np.cos(pos / (10000.0 ** (_2i / D_MODEL)))
    return pe


_PE_CONST = _positional_encoding_padded()


def _chunk_params(c):
    return c // 2, LEN_A * (c % 2), LEN_A if c % 2 == 0 else LEN_B


def _body(x_hbm, pe_hbm, table_hbm, out_hbm, idx_v, pe_v, rows_v, den_v,
          gsem, ssem, isem):
    wid = lax.axis_index("s") * NUM_CORES + lax.axis_index("c")
    seq0 = wid * SEQ_PER_W

    for u in range(SEQ_PER_W):
        pltpu.async_copy(x_hbm.at[seq0 + u, pl.ds(0, 128)],
                         idx_v.at[pl.ds(u * XPAD, 128)], isem)
        pltpu.async_copy(x_hbm.at[seq0 + u, pl.ds(128, 128)],
                         idx_v.at[pl.ds(u * XPAD + 128, 128)], isem)
    pltpu.sync_copy(pe_hbm, pe_v)
    for _ in range(2 * SEQ_PER_W):
        pltpu.make_async_copy(x_hbm.at[seq0, pl.ds(0, 128)],
                              idx_v.at[pl.ds(0, 128)], isem).wait()

    SEQ_PER_GROUP = NROWB // 2

    def start_gather(c, g, b):
        u, s0, ln = _chunk_params(c)
        pltpu.async_copy(
            table_hbm.at[idx_v.at[pl.ds((g * SEQ_PER_GROUP + u) * XPAD + s0,
                                        ln)]],
            rows_v.at[b, pl.ds(0, ln)], gsem.at[b])

    def wait_gather(c, b):
        _, _, ln = _chunk_params(c)
        pltpu.make_async_copy(
            table_hbm.at[idx_v.at[pl.ds(0, ln)]],
            rows_v.at[b, pl.ds(0, ln)], gsem.at[b]).wait()

    def start_store(c, g, d):
        u, s0, ln = _chunk_params(c)
        pltpu.async_copy(den_v.at[d, pl.ds(0, ln)],
                         out_hbm.at[seq0 + g * SEQ_PER_GROUP + u,
                                    pl.ds(s0, ln)], ssem.at[d])

    def wait_store(c, d):
        _, s0, ln = _chunk_params(c)
        pltpu.make_async_copy(den_v.at[d, pl.ds(0, ln)],
                              out_hbm.at[seq0, pl.ds(s0, ln)],
                              ssem.at[d]).wait()

    for c in range(LOOK):
        start_gather(c, 0, c)

    @pl.loop(0, NCHUNK // NROWB)
    def _group(g):
        for b in range(NROWB):
            c = b
            b2 = (b + LOOK) % NROWB
            d = b % NDENB

            @pl.when(g * NROWB + b + LOOK < NCHUNK)
            def _():
                start_gather(b + LOOK, g, b2)

            wait_gather(c, b)

            @pl.when(g * NROWB + b >= NDENB)
            def _():
                wait_store(c, d)

            u, s0, ln = _chunk_params(c)

            @pl.loop(0, ln, unroll=4)
            def _row(i):
                for j in range(D_MODEL // 16):
                    sl = pl.ds(j * 16, 16)
                    den_v[d, i, sl] = rows_v[b, i, sl] + pe_v[s0 + i, sl]

            start_store(c, g, d)

    for d in range(NDENB):
        wait_store(d, d)


@jax.jit
def _embed(x_p, pe, table_p):
    kfn = pl.kernel(
        _body,
        name="embed_gather",
        out_type=jax.ShapeDtypeStruct((BATCH, SEQ_LEN, D_MODEL), jnp.float32),
        mesh=plsc.VectorSubcoreMesh(core_axis_name="c", subcore_axis_name="s"),
        scratch_types=[
            pltpu.VMEM((SEQ_PER_W * XPAD,), jnp.int32),
            pltpu.VMEM((SEQ_LEN, DPAD), jnp.float32),
            pltpu.VMEM((NROWB, LEN_B, DPAD), jnp.float32),
            pltpu.VMEM((NDENB, LEN_B, D_MODEL), jnp.float32),
            pltpu.SemaphoreType.DMA((NROWB,)),
            pltpu.SemaphoreType.DMA((NDENB,)),
            pltpu.SemaphoreType.DMA,
        ],
    )
    return kfn(x_p, pe, table_p)


def kernel(x, table):
    pe = jnp.asarray(_PE_CONST)
    x_p = jnp.pad(x.astype(jnp.int32), ((0, 0), (0, XPAD - SEQ_LEN)))
    table_p = jnp.pad(table, ((0, 0), (0, DPAD - D_MODEL)))
    return _embed(x_p, pe, table_p)
